# baseline (speedup 1.0000x reference)
import functools

import jax
import jax.numpy as jnp
from jax import lax
from jax.experimental import pallas as pl
from jax.experimental.pallas import tpu as pltpu
from jax.experimental.pallas import tpu_sc as plsc


@functools.cache
def _make_embed_gather(V, D, BT, PT):
    B = BT * PT
    info = plsc.get_sparse_core_info()
    NC, NS = info.num_cores, info.num_subcores
    NW = NC * NS
    assert B % NW == 0
    b_per_w = B // NW
    assert PT % b_per_w == 0
    CH = 32
    NBUF = 5
    assert b_per_w % CH == 0 and CH <= 128
    NCH = b_per_w // CH

    mesh = plsc.VectorSubcoreMesh(core_axis_name="c", subcore_axis_name="s")

    @functools.partial(
        pl.kernel,
        mesh=mesh,
        out_type=jax.ShapeDtypeStruct((B, D), jnp.float32),
        scratch_types=[
            pltpu.VMEM((b_per_w,), jnp.int32),
        ]
        + [pltpu.VMEM((CH, D), jnp.float32)] * NBUF
        + [pltpu.SemaphoreType.DMA] * (2 * NBUF),
    )
    def k(idx_hbm, table_hbm, out_hbm, idx_v, *rest):
        bufs = rest[:NBUF]
        gsems = rest[NBUF : 2 * NBUF]
        wsems = rest[2 * NBUF :]

        wid = lax.axis_index("c") * NS + lax.axis_index("s")
        base = wid * b_per_w
        row = wid // (PT // b_per_w)
        col = (wid % (PT // b_per_w)) * b_per_w
        pltpu.sync_copy(idx_hbm.at[row, pl.ds(col, b_per_w)], idx_v)

        def gather(c):
            s = c % NBUF
            return pltpu.async_copy(
                table_hbm.at[idx_v.at[pl.ds(c * CH, CH)]], bufs[s], gsems[s]
            )

        K = NBUF - 1
        gathers = [None] * NCH
        writes = [None] * NCH
        for c in range(min(K, NCH)):
            gathers[c] = gather(c)
        for c in range(NCH):
            s = c % NBUF
            n = c + K
            if n < NCH:
                if n - NBUF >= 0:
                    writes[n - NBUF].wait()
                gathers[n] = gather(n)
            gathers[c].wait()
            writes[c] = pltpu.async_copy(
                bufs[s], out_hbm.at[pl.ds(base + c * CH, CH)], wsems[s]
            )
        for c in range(max(0, NCH - NBUF), NCH):
            writes[c].wait()

    return k


def kernel(tokens, W_E):
    BT, PT = tokens.shape
    V, D = W_E.shape
    out = _make_embed_gather(V, D, BT, PT)(tokens.astype(jnp.int32), W_E)
    return out.reshape(BT, PT, D)


# speedup vs baseline: 1.0077x; 1.0077x over previous
import functools

import jax
import jax.numpy as jnp
from jax import lax
from jax.experimental import pallas as pl
from jax.experimental.pallas import tpu as pltpu
from jax.experimental.pallas import tpu_sc as plsc


@functools.cache
def _make_embed_gather(V, D, BT, PT):
    B = BT * PT
    info = plsc.get_sparse_core_info()
    NC, NS = info.num_cores, info.num_subcores
    NW = NC * NS
    assert B % NW == 0
    b_per_w = B // NW
    assert PT % b_per_w == 0
    CH = 32
    NBUF = 5
    assert b_per_w % CH == 0 and CH <= 128
    NCH = b_per_w // CH

    mesh = plsc.VectorSubcoreMesh(core_axis_name="c", subcore_axis_name="s")

    @functools.partial(
        pl.kernel,
        mesh=mesh,
        out_type=jax.ShapeDtypeStruct((B, D), jnp.float32),
        scratch_types=[
            pltpu.VMEM((b_per_w,), jnp.int32),
        ]
        + [pltpu.VMEM((CH, D), jnp.float32)] * NBUF
        + [pltpu.SemaphoreType.DMA] * (2 * NBUF),
    )
    def k(idx_hbm, table_hbm, out_hbm, idx_v, *rest):
        bufs = rest[:NBUF]
        gsems = rest[NBUF : 2 * NBUF]
        wsems = rest[2 * NBUF :]

        wid = lax.axis_index("s") * NC + lax.axis_index("c")
        base = wid * b_per_w
        row = wid // (PT // b_per_w)
        col = (wid % (PT // b_per_w)) * b_per_w
        pltpu.sync_copy(idx_hbm.at[row, pl.ds(col, b_per_w)], idx_v)

        def gather(c):
            s = c % NBUF
            return pltpu.async_copy(
                table_hbm.at[idx_v.at[pl.ds(c * CH, CH)]], bufs[s], gsems[s]
            )

        K = NBUF - 1
        gathers = [None] * NCH
        writes = [None] * NCH
        for c in range(min(K, NCH)):
            gathers[c] = gather(c)
        for c in range(NCH):
            s = c % NBUF
            n = c + K
            if n < NCH:
                if n - NBUF >= 0:
                    writes[n - NBUF].wait()
                gathers[n] = gather(n)
            gathers[c].wait()
            writes[c] = pltpu.async_copy(
                bufs[s], out_hbm.at[pl.ds(base + c * CH, CH)], wsems[s]
            )
        for c in range(max(0, NCH - NBUF), NCH):
            writes[c].wait()

    return k


def kernel(tokens, W_E):
    BT, PT = tokens.shape
    V, D = W_E.shape
    out = _make_embed_gather(V, D, BT, PT)(tokens.astype(jnp.int32), W_E)
    return out.reshape(BT, PT, D)
